# Initial kernel scaffold; baseline (speedup 1.0000x reference)
import jax
import jax.numpy as jnp
from jax.experimental import pallas as pl


def kernel(x, modality_idx, embeddings):
    raise NotImplementedError("write your pallas kernel here")


# speedup vs baseline: 2.6002x; 2.6002x over previous
import functools

import jax
import jax.numpy as jnp
from jax import lax
from jax.experimental import pallas as pl
from jax.experimental.pallas import tpu as pltpu

B, S, D = 4, 4096, 1024
N = B * S
V = 8
BLK = 512
NBLK = N // BLK


def _tc_body(idx_ref, x_ref, emb_ref, o_ref):
    idx = idx_ref[0, 0, :]
    onehot = (idx[:, None] == lax.broadcasted_iota(jnp.int32, (1, V), 1)
              ).astype(jnp.float32)
    gathered = jnp.dot(onehot, emb_ref[...], preferred_element_type=jnp.float32)
    o_ref[...] = x_ref[...] + gathered


@jax.jit
def _tc_add(x2d, idx3d, emb):
    return pl.pallas_call(
        _tc_body,
        grid=(NBLK,),
        in_specs=[
            pl.BlockSpec((1, 1, BLK), lambda i: (i, 0, 0)),
            pl.BlockSpec((BLK, D), lambda i: (i, 0)),
            pl.BlockSpec((V, D), lambda i: (0, 0)),
        ],
        out_specs=pl.BlockSpec((BLK, D), lambda i: (i, 0)),
        out_shape=jax.ShapeDtypeStruct((N, D), jnp.float32),
    )(idx3d, x2d, emb)


def kernel(x, modality_idx, embeddings):
    x2d = x.reshape(N, D)
    idx3d = modality_idx.astype(jnp.int32).reshape(NBLK, 1, BLK)
    out = _tc_add(x2d, idx3d, embeddings)
    return out.reshape(B, S, D)
